# baseline (speedup 1.0000x reference)
import functools

import jax
import jax.numpy as jnp
from jax import lax
from jax.experimental import pallas as pl
from jax.experimental.pallas import tpu as pltpu
from jax.experimental.pallas import tpu_sc as plsc

E = 3_200_000
N = 100_000
G = 64
H = 128
L = 32
NC, NS = 2, 16
NW = NC * NS
EPW = E // NW
NBLK = E // 128
NBT = 832
CB = 64
NCHK = 13
RING = 4 * CB
K = 50
NKB = EPW // (16 * K)
NPW = 3_200
NPAD = NW * NPW - N
GP = 80

_mesh = plsc.VectorSubcoreMesh(core_axis_name="c", subcore_axis_name="s")


@functools.partial(
    pl.kernel,
    out_type=(
        tuple(jax.ShapeDtypeStruct((NW, 16 * G), jnp.float32)
              for _ in range(5))
        + (jax.ShapeDtypeStruct((NW, 16 * GP), jnp.float32),)
    ),
    mesh=_mesh,
    scratch_types=(
        [pltpu.VMEM((RING * 256,), jnp.float32)]
        + [pltpu.VMEM((RING * 128,), jnp.int32)]
        + [pltpu.VMEM((NPW,), jnp.int32)]
        + [pltpu.VMEM((16 * G,), jnp.float32) for _ in range(5)]
        + [pltpu.VMEM((16 * GP,), jnp.float32)]
        + [pltpu.SemaphoreType.DMA for _ in range(9)]
    ),
    compiler_params=pltpu.CompilerParams(needs_layout_passes=False),
)
def _sc_segment(eb_hbm, exf_hbm, nb_hbm,
                o_sum, o_cnt, o_knw, o_max, o_evt, nt_out,
                xr, er, nbb,
                tsum, tcnt, tknw, tmax, tevt, accn,
                sx0, sx1, sx2, sx3, sb0, sb1, sb2, sb3, sn):
    wid = lax.axis_index("s") * NC + lax.axis_index("c")
    lane = lax.iota(jnp.int32, 16)
    lane_g = lane * G
    lane_gp = lane * GP
    zeros = jnp.zeros((16,), jnp.float32)
    ones = jnp.ones((16,), jnp.float32)

    tabs = (tsum, tcnt, tknw, tmax, tevt)

    def zrow(r, carry):
        for t in tabs:
            t[pl.ds(r * 16, 16)] = zeros
        return carry
    lax.fori_loop(0, G, zrow, 0)

    def zrow_n(r, carry):
        accn[pl.ds(r * 16, 16)] = zeros
        return carry
    lax.fori_loop(0, GP, zrow_n, 0)

    ebase = wid * EPW
    b0 = jnp.minimum(ebase // 128, NBLK - NBT)
    nl = (ebase - b0 * 128) // 16
    sx = (sx0, sx1, sx2, sx3)
    sb = (sb0, sb1, sb2, sb3)

    def chunk_copies(c, slot):
        blk = (b0 + c * CB) * 256
        blkb = (b0 + c * CB) * 128
        return (pltpu.make_async_copy(
                    exf_hbm.at[pl.ds(blk, CB * 256)],
                    xr.at[pl.ds(slot * (CB * 256), CB * 256)], sx[slot]),
                pltpu.make_async_copy(
                    eb_hbm.at[pl.ds(blkb, CB * 128)],
                    er.at[pl.ds(slot * (CB * 128), CB * 128)], sb[slot]))

    def start_chunk(c, slot):
        for cp in chunk_copies(c, slot):
            cp.start()

    def wait_chunk(c, slot):
        for cp in chunk_copies(c, slot):
            cp.wait()

    start_chunk(0, 0)
    start_chunk(1, 1)
    start_chunk(2, 2)

    pltpu.async_copy(nb_hbm.at[pl.ds(wid * NPW, NPW)], nbb, sn).wait()

    def nstep(i, carry):
        b = nbb[pl.ds(i * 16, 16)]
        plsc.addupdate_scatter(accn, [lane_gp + b], ones)
        return carry
    lax.fori_loop(0, NPW // 16, nstep, 0)

    def flush(bprev, rs, rc, rk, rm, re):
        bi = lane_g + bprev
        plsc.addupdate_scatter(tsum, [bi], rs)
        plsc.addupdate_scatter(tcnt, [bi], rc)
        plsc.addupdate_scatter(tknw, [bi], rk)
        m = plsc.load_gather(tmax, [bi])
        plsc.store_scatter(tmax, [bi], jnp.maximum(m, rm))
        e_ = plsc.load_gather(tevt, [bi])
        plsc.store_scatter(tevt, [bi], jnp.maximum(e_, re))

    def addrs(s):
        t = s << 4
        bo = t & (RING * 128 - 1)
        xo = ((bo & ~jnp.int32(127)) << 1) + (bo & 127)
        return xo, bo

    def load_b(s):
        _, bo = addrs(s)
        return er[pl.ds(bo, 16)]

    def load_ao(s):
        xo, _ = addrs(s)
        return xr[pl.ds(xo, 16)], xr[pl.ds(xo + 128, 16)]

    kf = jnp.float32(K)

    def kblock(j, c):
        bprev0 = c[0]
        base = nl + j * K
        b_first = load_b(base)
        b_last = load_b(base + (K - 1))
        uni = jnp.logical_and(jnp.all(b_first == bprev0),
                              jnp.all(b_last == bprev0))

        def fast(c):
            bp, rs0, rs1, rc0, rc1, rk0, rk1, rm0, rm1, re0, re1 = c
            r = [rs0, rs1, rk0, rk1, rm0, rm1, re0, re1]
            for u in range(K):
                a, o = load_ao(base + u)
                kn = o >= 0.0
                p = u & 1
                r[0 + p] = r[0 + p] + a
                r[2 + p] = r[2 + p] + jnp.where(kn, ones, zeros)
                r[4 + p] = jnp.maximum(r[4 + p], a)
                r[6 + p] = jnp.maximum(r[6 + p],
                                       jnp.where(kn, o + 1.0, zeros))
            return (bp, r[0], r[1], rc0 + kf, rc1,
                    r[2], r[3], r[4], r[5], r[6], r[7])

        def slow(c):
            bprev, rs0, rs1, rc0, rc1, rk0, rk1, rm0, rm1, re0, re1 = c
            rs, rc = rs0 + rs1, rc0 + rc1
            rk = rk0 + rk1
            rm, re = jnp.maximum(rm0, rm1), jnp.maximum(re0, re1)
            for u in range(K):
                b = load_b(base + u)
                a, o = load_ao(base + u)
                chg = b != bprev
                bi = lane_g + bprev
                plsc.addupdate_scatter(tsum, [bi], rs, mask=chg)
                plsc.addupdate_scatter(tcnt, [bi], rc, mask=chg)
                plsc.addupdate_scatter(tknw, [bi], rk, mask=chg)
                m = plsc.load_gather(tmax, [bi])
                plsc.store_scatter(tmax, [bi], jnp.maximum(m, rm), mask=chg)
                e_ = plsc.load_gather(tevt, [bi])
                plsc.store_scatter(tevt, [bi], jnp.maximum(e_, re), mask=chg)
                gone = jnp.where(chg, zeros, ones)
                kn = o >= 0.0
                rs = rs * gone + a
                rc = rc * gone + ones
                rk = rk * gone + jnp.where(kn, ones, zeros)
                rm = jnp.maximum(rm * gone, a)
                re = jnp.maximum(re * gone, jnp.where(kn, o + 1.0, zeros))
                bprev = b
            return (bprev, rs, zeros, rc, zeros,
                    rk, zeros, rm, zeros, re, zeros)

        return lax.cond(uni, fast, slow, c)

    def range_for(c, phase, state):
        jprev = state[0]
        carry = state[1:]
        wait_chunk(c, phase)
        jhi = jnp.minimum(jnp.int32(NKB),
                          (512 * (c + 1) - K - nl) // K + 1)
        carry = lax.fori_loop(jprev, jhi, kblock, carry)
        return (jhi,) + tuple(carry)

    state = (jnp.int32(0), jnp.zeros((16,), jnp.int32)) + (zeros,) * 10

    def round4(r, st):
        for phase in range(4):
            c = r * 4 + phase
            st = range_for(c, phase, st)

            @pl.when(c + 3 < NCHK)
            def _():
                start_chunk(c + 3, (phase + 3) % 4)
        return st

    state = lax.fori_loop(0, (NCHK - 1) // 4, round4, state)
    state = range_for(jnp.int32(NCHK - 1), (NCHK - 1) % 4, state)

    _, bp, rs0, rs1, rc0, rc1, rk0, rk1, rm0, rm1, re0, re1 = state
    flush(bp, rs0 + rs1, rc0 + rc1, rk0 + rk1,
          jnp.maximum(rm0, rm1), jnp.maximum(re0, re1))

    outs = (o_sum, o_cnt, o_knw, o_max, o_evt)
    for t, ot in zip(tabs, outs):
        pltpu.sync_copy(t, ot.at[wid])
    pltpu.sync_copy(accn, nt_out.at[wid])


def _finish_body(ts_ref, tc_ref, tk_ref, tm_ref, te_ref, nt_ref,
                 w1t_ref, b1_ref, w2t_ref, b2_ref, out_ref):
    def rsum(ref, stride):
        x = ref[...]
        acc = x[:, 0:G]
        for l in range(1, 16):
            acc = acc + x[:, l * stride:l * stride + G]
        return acc.sum(axis=0)

    def rmax(ref, stride):
        x = ref[...]
        acc = x[:, 0:G]
        for l in range(1, 16):
            acc = jnp.maximum(acc, x[:, l * stride:l * stride + G])
        return acc.max(axis=0)

    nn = rsum(nt_ref, GP)
    s_ = rsum(ts_ref, G)
    c_ = rsum(tc_ref, G)
    k_ = rsum(tk_ref, G)
    m_ = rmax(tm_ref, G)
    ev = rmax(te_ref, G)
    denom = jnp.maximum(c_, 1.0)
    feats_t = jnp.concatenate(
        [nn[None], c_[None], (s_ / denom)[None], m_[None],
         (k_ / denom)[None], ev[None]], axis=0)
    h_t = jnp.maximum(
        jnp.dot(w1t_ref[...], feats_t,
                preferred_element_type=jnp.float32) + b1_ref[...], 0.0)
    out = lax.dot_general(
        h_t, w2t_ref[...], (((0,), (1,)), ((), ())),
        preferred_element_type=jnp.float32)
    out_ref[...] = out + b2_ref[...]


def kernel(node_x, edge_x, node_batch, edge_batch, W1, b1, W2, b2):
    exf = edge_x.reshape(NBLK, 128, 2).transpose(0, 2, 1).reshape(2 * E)
    nb = jnp.concatenate(
        [node_batch, jnp.full((NPAD,), G, jnp.int32)])
    *tables, nt = _sc_segment(edge_batch, exf, nb)
    return pl.pallas_call(
        _finish_body,
        out_shape=jax.ShapeDtypeStruct((G, L), jnp.float32),
    )(*tables, nt, W1.T, b1.reshape(H, 1), W2.T, b2.reshape(1, L))


# speedup vs baseline: 1.0555x; 1.0555x over previous
import functools

import jax
import jax.numpy as jnp
from jax import lax
from jax.experimental import pallas as pl
from jax.experimental.pallas import tpu as pltpu
from jax.experimental.pallas import tpu_sc as plsc

E = 3_200_000
N = 100_000
G = 64
H = 128
L = 32
NC, NS = 2, 16
NW = NC * NS
EPW = E // NW
NBLK = E // 128
NBT = 832
CB = 64
NCHK = 13
RING = 4 * CB
K = 25
NKB = EPW // (16 * K)
NPW = 3_200
NPAD = NW * NPW - N
GP = 80

_mesh = plsc.VectorSubcoreMesh(core_axis_name="c", subcore_axis_name="s")


@functools.partial(
    pl.kernel,
    out_type=(
        tuple(jax.ShapeDtypeStruct((NW, 16 * G), jnp.float32)
              for _ in range(5))
        + (jax.ShapeDtypeStruct((NW, 16 * GP), jnp.float32),)
    ),
    mesh=_mesh,
    scratch_types=(
        [pltpu.VMEM((RING * 256,), jnp.float32)]
        + [pltpu.VMEM((RING * 128,), jnp.int32)]
        + [pltpu.VMEM((NPW,), jnp.int32)]
        + [pltpu.VMEM((16 * G,), jnp.float32) for _ in range(5)]
        + [pltpu.VMEM((16 * GP,), jnp.float32)]
        + [pltpu.SemaphoreType.DMA for _ in range(9)]
    ),
    compiler_params=pltpu.CompilerParams(needs_layout_passes=False),
)
def _sc_segment(eb_hbm, exf_hbm, nb_hbm,
                o_sum, o_cnt, o_knw, o_max, o_evt, nt_out,
                xr, er, nbb,
                tsum, tcnt, tknw, tmax, tevt, accn,
                sx0, sx1, sx2, sx3, sb0, sb1, sb2, sb3, sn):
    wid = lax.axis_index("s") * NC + lax.axis_index("c")
    lane = lax.iota(jnp.int32, 16)
    lane_g = lane * G
    lane_gp = lane * GP
    zeros = jnp.zeros((16,), jnp.float32)
    ones = jnp.ones((16,), jnp.float32)

    tabs = (tsum, tcnt, tknw, tmax, tevt)

    def zrow(r, carry):
        for t in tabs:
            t[pl.ds(r * 16, 16)] = zeros
        return carry
    lax.fori_loop(0, G, zrow, 0)

    def zrow_n(r, carry):
        accn[pl.ds(r * 16, 16)] = zeros
        return carry
    lax.fori_loop(0, GP, zrow_n, 0)

    ebase = wid * EPW
    b0 = jnp.minimum(ebase // 128, NBLK - NBT)
    nl = (ebase - b0 * 128) // 16
    sx = (sx0, sx1, sx2, sx3)
    sb = (sb0, sb1, sb2, sb3)

    def chunk_copies(c, slot):
        blk = (b0 + c * CB) * 256
        blkb = (b0 + c * CB) * 128
        return (pltpu.make_async_copy(
                    exf_hbm.at[pl.ds(blk, CB * 256)],
                    xr.at[pl.ds(slot * (CB * 256), CB * 256)], sx[slot]),
                pltpu.make_async_copy(
                    eb_hbm.at[pl.ds(blkb, CB * 128)],
                    er.at[pl.ds(slot * (CB * 128), CB * 128)], sb[slot]))

    def start_chunk(c, slot):
        for cp in chunk_copies(c, slot):
            cp.start()

    def wait_chunk(c, slot):
        for cp in chunk_copies(c, slot):
            cp.wait()

    start_chunk(0, 0)
    start_chunk(1, 1)
    start_chunk(2, 2)

    pltpu.async_copy(nb_hbm.at[pl.ds(wid * NPW, NPW)], nbb, sn).wait()

    def nstep(i, carry):
        b = nbb[pl.ds(i * 16, 16)]
        plsc.addupdate_scatter(accn, [lane_gp + b], ones)
        return carry
    lax.fori_loop(0, NPW // 16, nstep, 0)

    def flush(bprev, rs, rc, rk, rm, re):
        bi = lane_g + bprev
        plsc.addupdate_scatter(tsum, [bi], rs)
        plsc.addupdate_scatter(tcnt, [bi], rc)
        plsc.addupdate_scatter(tknw, [bi], rk)
        m = plsc.load_gather(tmax, [bi])
        plsc.store_scatter(tmax, [bi], jnp.maximum(m, rm))
        e_ = plsc.load_gather(tevt, [bi])
        plsc.store_scatter(tevt, [bi], jnp.maximum(e_, re))

    def addrs(s):
        t = s << 4
        bo = t & (RING * 128 - 1)
        xo = ((bo & ~jnp.int32(127)) << 1) + (bo & 127)
        return xo, bo

    def load_b(s):
        _, bo = addrs(s)
        return er[pl.ds(bo, 16)]

    def load_ao(s):
        xo, _ = addrs(s)
        return xr[pl.ds(xo, 16)], xr[pl.ds(xo + 128, 16)]

    kf = jnp.float32(K)

    def kblock(j, c):
        bprev0 = c[0]
        base = nl + j * K
        b_first = load_b(base)
        b_last = load_b(base + (K - 1))
        uni = jnp.all(jnp.logical_and(b_first == bprev0,
                                      b_last == bprev0))

        def fast(c):
            bp, rs0, rs1, rc0, rc1, rk0, rk1, rm0, rm1, re0, re1 = c
            r = [rs0, rs1, rk0, rk1, rm0, rm1, re0, re1]
            for u in range(K):
                a, o = load_ao(base + u)
                kn = o >= 0.0
                p = u & 1
                r[0 + p] = r[0 + p] + a
                r[2 + p] = r[2 + p] + jnp.where(kn, ones, zeros)
                r[4 + p] = jnp.maximum(r[4 + p], a)
                r[6 + p] = jnp.maximum(r[6 + p],
                                       jnp.where(kn, o + 1.0, zeros))
            return (bp, r[0], r[1], rc0 + kf, rc1,
                    r[2], r[3], r[4], r[5], r[6], r[7])

        def slow(c):
            bprev, rs0, rs1, rc0, rc1, rk0, rk1, rm0, rm1, re0, re1 = c
            rs, rc = rs0 + rs1, rc0 + rc1
            rk = rk0 + rk1
            rm, re = jnp.maximum(rm0, rm1), jnp.maximum(re0, re1)
            for u in range(K):
                b = load_b(base + u)
                a, o = load_ao(base + u)
                chg = b != bprev
                bi = lane_g + bprev
                plsc.addupdate_scatter(tsum, [bi], rs, mask=chg)
                plsc.addupdate_scatter(tcnt, [bi], rc, mask=chg)
                plsc.addupdate_scatter(tknw, [bi], rk, mask=chg)
                m = plsc.load_gather(tmax, [bi])
                plsc.store_scatter(tmax, [bi], jnp.maximum(m, rm), mask=chg)
                e_ = plsc.load_gather(tevt, [bi])
                plsc.store_scatter(tevt, [bi], jnp.maximum(e_, re), mask=chg)
                gone = jnp.where(chg, zeros, ones)
                kn = o >= 0.0
                rs = rs * gone + a
                rc = rc * gone + ones
                rk = rk * gone + jnp.where(kn, ones, zeros)
                rm = jnp.maximum(rm * gone, a)
                re = jnp.maximum(re * gone, jnp.where(kn, o + 1.0, zeros))
                bprev = b
            return (bprev, rs, zeros, rc, zeros,
                    rk, zeros, rm, zeros, re, zeros)

        return lax.cond(uni, fast, slow, c)

    def range_for(c, phase, state):
        jprev = state[0]
        carry = state[1:]
        wait_chunk(c, phase)
        jhi = jnp.minimum(jnp.int32(NKB),
                          (512 * (c + 1) - 25 - nl) // 25 + 1)
        carry = lax.fori_loop(jprev, jhi, kblock, carry)
        return (jhi,) + tuple(carry)

    state = (jnp.int32(0), jnp.zeros((16,), jnp.int32)) + (zeros,) * 10

    def round4(r, st):
        for phase in range(4):
            c = r * 4 + phase
            st = range_for(c, phase, st)

            @pl.when(c + 3 < NCHK)
            def _():
                start_chunk(c + 3, (phase + 3) % 4)
        return st

    state = lax.fori_loop(0, (NCHK - 1) // 4, round4, state)
    state = range_for(jnp.int32(NCHK - 1), (NCHK - 1) % 4, state)

    _, bp, rs0, rs1, rc0, rc1, rk0, rk1, rm0, rm1, re0, re1 = state
    flush(bp, rs0 + rs1, rc0 + rc1, rk0 + rk1,
          jnp.maximum(rm0, rm1), jnp.maximum(re0, re1))

    outs = (o_sum, o_cnt, o_knw, o_max, o_evt)
    for t, ot in zip(tabs, outs):
        pltpu.sync_copy(t, ot.at[wid])
    pltpu.sync_copy(accn, nt_out.at[wid])


def _finish_body(ts_ref, tc_ref, tk_ref, tm_ref, te_ref, nt_ref,
                 w1t_ref, b1_ref, w2t_ref, b2_ref, out_ref):
    def rsum(ref, stride):
        x = ref[...]
        acc = x[:, 0:G]
        for l in range(1, 16):
            acc = acc + x[:, l * stride:l * stride + G]
        return acc.sum(axis=0)

    def rmax(ref, stride):
        x = ref[...]
        acc = x[:, 0:G]
        for l in range(1, 16):
            acc = jnp.maximum(acc, x[:, l * stride:l * stride + G])
        return acc.max(axis=0)

    nn = rsum(nt_ref, GP)
    s_ = rsum(ts_ref, G)
    c_ = rsum(tc_ref, G)
    k_ = rsum(tk_ref, G)
    m_ = rmax(tm_ref, G)
    ev = rmax(te_ref, G)
    denom = jnp.maximum(c_, 1.0)
    feats_t = jnp.concatenate(
        [nn[None], c_[None], (s_ / denom)[None], m_[None],
         (k_ / denom)[None], ev[None]], axis=0)
    h_t = jnp.maximum(
        jnp.dot(w1t_ref[...], feats_t,
                preferred_element_type=jnp.float32) + b1_ref[...], 0.0)
    out = lax.dot_general(
        h_t, w2t_ref[...], (((0,), (1,)), ((), ())),
        preferred_element_type=jnp.float32)
    out_ref[...] = out + b2_ref[...]


def kernel(node_x, edge_x, node_batch, edge_batch, W1, b1, W2, b2):
    exf = edge_x.reshape(NBLK, 128, 2).transpose(0, 2, 1).reshape(2 * E)
    nb = jnp.concatenate(
        [node_batch, jnp.full((NPAD,), G, jnp.int32)])
    *tables, nt = _sc_segment(edge_batch, exf, nb)
    return pl.pallas_call(
        _finish_body,
        out_shape=jax.ShapeDtypeStruct((G, L), jnp.float32),
    )(*tables, nt, W1.T, b1.reshape(H, 1), W2.T, b2.reshape(1, L))
